# baseline (speedup 1.0000x reference)
import functools

import jax
import jax.numpy as jnp
from jax import lax
from jax.experimental import pallas as pl
from jax.experimental.pallas import tpu as pltpu
from jax.experimental.pallas import tpu_sc as plsc

N = 10000
E = 160000
D_IN = 128
H1 = 640
H2 = 320
D_OUT = 128

N_PAD = 10240
E_PAD = 163840
CHUNK = 64
CH_PER_TILE = 80
EPT = CHUNK * CH_PER_TILE
ROWS_PER_TILE = N_PAD // 16
NBUF = 3

_MESH = plsc.VectorSubcoreMesh(core_axis_name="c", subcore_axis_name="s")


def _elu(v):
    return jnp.where(v > 0, v, jnp.exp(v) - 1.0)


@functools.partial(
    pl.kernel,
    out_type=jax.ShapeDtypeStruct((2, 2, N_PAD, 16), jnp.float32),
    mesh=_MESH,
    compiler_params=pltpu.CompilerParams(use_tc_tiling_on_sc=False),
    scratch_types=[
        pltpu.VMEM((CH_PER_TILE, CHUNK), jnp.int32),
        pltpu.VMEM((CH_PER_TILE, CHUNK), jnp.int32),
        pltpu.VMEM((CHUNK, 16), jnp.float32),
        pltpu.VMEM_SHARED((N_PAD, 16), jnp.float32),
        pltpu.VMEM_SHARED((N_PAD, 16), jnp.float32),
    ],
)
def _sc_degrees(src_hbm, dst_hbm, out_hbm,
                src_v, dst_v, fill_v, ds_sh, dd_sh):
    c = lax.axis_index("c")
    s = lax.axis_index("s")
    wid = c * 16 + s

    pltpu.sync_copy(src_hbm.at[pl.ds(wid * CH_PER_TILE, CH_PER_TILE)], src_v)
    pltpu.sync_copy(dst_hbm.at[pl.ds(wid * CH_PER_TILE, CH_PER_TILE)], dst_v)

    def _fill(val):
        vec = jnp.full((16,), val, jnp.float32)

        def _frow(r, _):
            fill_v[r, pl.ds(0, 16)] = vec
            return 0

        lax.fori_loop(0, CHUNK, _frow, 0)

    _fill(0.0)
    for z in range(ROWS_PER_TILE // CHUNK):
        r0 = s * ROWS_PER_TILE + z * CHUNK
        pltpu.sync_copy(fill_v, ds_sh.at[pl.ds(r0, CHUNK)])
        pltpu.sync_copy(fill_v, dd_sh.at[pl.ds(r0, CHUNK)])
    _fill(1.0)
    plsc.subcore_barrier()

    def _edge_chunk(j, _):
        pltpu.sync_copy(fill_v, ds_sh.at[src_v.at[j]], add=True)
        pltpu.sync_copy(fill_v, dd_sh.at[dst_v.at[j]], add=True)
        return 0

    lax.fori_loop(0, CH_PER_TILE, _edge_chunk, 0)
    plsc.subcore_barrier()

    rows = pl.ds(s * ROWS_PER_TILE, ROWS_PER_TILE)
    pltpu.sync_copy(ds_sh.at[rows], out_hbm.at[c, 0, rows])
    pltpu.sync_copy(dd_sh.at[rows], out_hbm.at[c, 1, rows])


def _make_sc_agg(n_tab):
    w = 128

    @functools.partial(
        pl.kernel,
        out_type=jax.ShapeDtypeStruct((n_tab, 2, N_PAD, w), jnp.float32),
        mesh=_MESH,
        scratch_types=[
            pltpu.VMEM((CH_PER_TILE, CHUNK), jnp.int32),
            pltpu.VMEM((CH_PER_TILE, CHUNK), jnp.int32),
            pltpu.VMEM((NBUF, CHUNK, w), jnp.float32),
            pltpu.VMEM_SHARED((N_PAD, w), jnp.float32),
            pltpu.SemaphoreType.DMA((NBUF,)),
            pltpu.SemaphoreType.DMA((NBUF,)),
        ],
    )
    def _sc_agg(h_hbm, src_hbm, dst_hbm, out_hbm,
                src_v, dst_v, rows_v, acc_sh, gsems, ssems):
        c = lax.axis_index("c")
        s = lax.axis_index("s")
        wid = c * 16 + s

        pltpu.sync_copy(src_hbm.at[pl.ds(wid * CH_PER_TILE, CH_PER_TILE)],
                        src_v)
        pltpu.sync_copy(dst_hbm.at[pl.ds(wid * CH_PER_TILE, CH_PER_TILE)],
                        dst_v)

        zero16 = jnp.zeros((16,), jnp.float32)

        def _zrow(r, _):
            def _zcol(q, _):
                rows_v[0, r, pl.ds(q * 16, 16)] = zero16
                return 0
            lax.fori_loop(0, w // 16, _zcol, 0)
            return 0

        lax.fori_loop(0, CHUNK, _zrow, 0)

        my_rows = pl.ds(s * ROWS_PER_TILE, ROWS_PER_TILE)

        for t in range(n_tab):
            for z in range(ROWS_PER_TILE // CHUNK):
                r0 = s * ROWS_PER_TILE + z * CHUNK
                pltpu.sync_copy(rows_v.at[0], acc_sh.at[pl.ds(r0, CHUNK)])
            plsc.subcore_barrier()

            def _burst(base, nb):
                gd = [pltpu.async_copy(h_hbm.at[t].at[src_v.at[base + b]],
                                       rows_v.at[b], gsems.at[b])
                      for b in range(nb)]
                sd = []
                for b in range(nb):
                    gd[b].wait()
                    sd.append(pltpu.async_copy(
                        rows_v.at[b], acc_sh.at[dst_v.at[base + b]],
                        ssems.at[b], add=True))
                for b in range(nb):
                    sd[b].wait()

            def _step(st, _):
                _burst(st * NBUF, NBUF)
                return 0

            n_steps = CH_PER_TILE // NBUF
            lax.fori_loop(0, n_steps, _step, 0)
            if CH_PER_TILE % NBUF:
                _burst(n_steps * NBUF, CH_PER_TILE % NBUF)
            plsc.subcore_barrier()

            pltpu.sync_copy(acc_sh.at[my_rows], out_hbm.at[t, c, my_rows])
            if t + 1 < n_tab:
                plsc.subcore_barrier()

            if t + 1 < n_tab:
                lax.fori_loop(0, CHUNK, _zrow, 0)

    return _sc_agg


_sc_agg1 = _make_sc_agg(1)
_sc_agg3 = _make_sc_agg(3)


BN = 1000
GRID = N // BN

_row_spec = lambda wdt: pl.BlockSpec((BN, wdt), lambda i: (i, 0))
_vec_spec = pl.BlockSpec((BN, 1), lambda i: (i, 0))
_h1_spec = pl.BlockSpec((1, BN, D_IN), lambda i: (0, i, 0))
_y2_spec = pl.BlockSpec((3, BN, 128), lambda i: (0, i, 0))
_p2_spec = pl.BlockSpec((2, BN, 128), lambda i: (0, i, 0))
_p32_spec = pl.BlockSpec((3, 2, BN, 128), lambda i: (0, 0, i, 0))
_deg_spec = pl.BlockSpec((2, BN, 1), lambda i: (0, i, 0))


def _full(shape):
    nd = len(shape)
    return pl.BlockSpec(shape, lambda i: (0,) * nd)


def _tc0_body(f_ref, od_ref, id_ref, wres_ref, bres_ref,
              ns_ref, nd_ref, h1_ref, res_ref):
    od = od_ref[0] + od_ref[1]
    ig = id_ref[0] + id_ref[1]
    ns = lax.rsqrt(jnp.where(od > 0, od, 1.0))
    nd = lax.rsqrt(jnp.where(ig > 0, ig, 1.0))
    ns_ref[...] = ns
    nd_ref[...] = nd
    f = f_ref[...]
    h1_ref[0] = f * ns
    r = jnp.dot(f, wres_ref[...], preferred_element_type=jnp.float32)
    res_ref[...] = _elu(r + bres_ref[...][None, :])


def _tc0(f, od2, id2, Wres, bres):
    return pl.pallas_call(
        _tc0_body,
        grid=(GRID,),
        in_specs=[_row_spec(D_IN), _deg_spec, _deg_spec,
                  _full((D_IN, D_OUT)), _full((D_OUT,))],
        out_specs=[_vec_spec, _vec_spec, _h1_spec, _row_spec(D_OUT)],
        out_shape=[
            jax.ShapeDtypeStruct((N, 1), jnp.float32),
            jax.ShapeDtypeStruct((N, 1), jnp.float32),
            jax.ShapeDtypeStruct((1, N, D_IN), jnp.float32),
            jax.ShapeDtypeStruct((N, D_OUT), jnp.float32),
        ],
    )(f, od2, id2, Wres, bres)


def _tc1_body(p_ref, nd_ref, ns_ref, w1_ref, b1_ref, w2s_ref, y2_ref):
    a1 = (p_ref[0] + p_ref[1]) * nd_ref[...]
    x1 = _elu(jnp.dot(a1, w1_ref[...], preferred_element_type=jnp.float32)
              + b1_ref[...][None, :])
    x1n = x1 * ns_ref[...]
    for t in range(3):
        y2_ref[t] = jnp.dot(x1n, w2s_ref[t],
                            preferred_element_type=jnp.float32)


def _tc1(p1, nd, ns, W1, b1, W2s):
    return pl.pallas_call(
        _tc1_body,
        grid=(GRID,),
        in_specs=[_p2_spec, _vec_spec, _vec_spec,
                  _full((D_IN, H1)), _full((H1,)), _full((3, H1, 128))],
        out_specs=[_y2_spec],
        out_shape=[jax.ShapeDtypeStruct((3, N, 128), jnp.float32)],
    )(p1, nd, ns, W1, b1, W2s)[0]


def _tc2_body(p_ref, nd_ref, ns_ref, b2s_ref, w3s_ref, y3_ref):
    nd = nd_ref[...]
    ns = ns_ref[...]
    acc = None
    for t in range(3):
        x2 = _elu((p_ref[t, 0] + p_ref[t, 1]) * nd + b2s_ref[t][None, :])
        d = jnp.dot(x2 * ns, w3s_ref[t], preferred_element_type=jnp.float32)
        acc = d if acc is None else acc + d
    y3_ref[0] = acc


def _tc2(p2, nd, ns, b2s, W3s):
    return pl.pallas_call(
        _tc2_body,
        grid=(GRID,),
        in_specs=[_p32_spec, _vec_spec, _vec_spec,
                  _full((3, 128)), _full((3, 128, D_OUT))],
        out_specs=[_h1_spec],
        out_shape=[jax.ShapeDtypeStruct((1, N, D_OUT), jnp.float32)],
    )(p2, nd, ns, b2s, W3s)[0]


def _tc3_body(p_ref, nd_ref, b3_ref, out_ref):
    out_ref[...] = ((p_ref[0] + p_ref[1]) * nd_ref[...]
                    + b3_ref[...][None, :])


def _tc3(p3, nd, b3):
    return pl.pallas_call(
        _tc3_body,
        grid=(GRID,),
        in_specs=[_p2_spec, _vec_spec, _full((D_OUT,))],
        out_specs=[_row_spec(D_OUT)],
        out_shape=[jax.ShapeDtypeStruct((N, D_OUT), jnp.float32)],
    )(p3, nd, b3)[0]


def kernel(features, edge_index, W1, b1, W2, b2, W3, b3, Wres, bres):
    pad_e = E_PAD - E
    dummy = N + (jnp.arange(pad_e, dtype=jnp.int32) % (N_PAD - N))
    src_r = edge_index[0].astype(jnp.int32)
    dst_r = edge_index[1].astype(jnp.int32)
    shape2 = (E_PAD // CHUNK, CHUNK)
    src_deg = jnp.concatenate([src_r, dummy]).reshape(shape2)
    src_agg = jnp.concatenate(
        [src_r, jnp.arange(pad_e, dtype=jnp.int32) % N]).reshape(shape2)
    dst = jnp.concatenate([dst_r, dummy]).reshape(shape2)

    deg = _sc_degrees(src_deg, dst)
    od2 = deg[:, 0, :, :1]
    id2 = deg[:, 1, :, :1]

    ns, nd, h1, res = _tc0(features, od2, id2, Wres, bres)

    p1 = _sc_agg1(h1, src_agg, dst)
    W2s = jnp.stack([W2[:, :128], W2[:, 128:256],
                     jnp.pad(W2[:, 256:], ((0, 0), (0, 64)))])
    y2 = _tc1(p1[0], nd, ns, W1, b1, W2s)

    p2 = _sc_agg3(y2, src_agg, dst)
    b2s = jnp.stack([b2[:128], b2[128:256], jnp.pad(b2[256:], (0, 64))])
    W3s = jnp.stack([W3[:128], W3[128:256],
                     jnp.pad(W3[256:], ((0, 64), (0, 0)))])
    y3 = _tc2(p2, nd, ns, b2s, W3s)

    p3 = _sc_agg1(y3, src_agg, dst)
    x = _tc3(p3[0], nd, b3)
    return (x, res)


# speedup vs baseline: 1.0384x; 1.0384x over previous
import functools

import jax
import jax.numpy as jnp
from jax import lax
from jax.experimental import pallas as pl
from jax.experimental.pallas import tpu as pltpu
from jax.experimental.pallas import tpu_sc as plsc

N = 10000
E = 160000
D_IN = 128
H1 = 640
H2 = 320
D_OUT = 128

N_PAD = 10240
E_PAD = 163840
CHUNK = 128
CH_PER_TILE = 40
EPT = CHUNK * CH_PER_TILE
ROWS_PER_TILE = N_PAD // 16
NBUF = 2

_MESH = plsc.VectorSubcoreMesh(core_axis_name="c", subcore_axis_name="s")


def _elu(v):
    return jnp.where(v > 0, v, jnp.exp(v) - 1.0)


@functools.partial(
    pl.kernel,
    out_type=jax.ShapeDtypeStruct((2, 2, N_PAD, 16), jnp.float32),
    mesh=_MESH,
    compiler_params=pltpu.CompilerParams(use_tc_tiling_on_sc=False),
    scratch_types=[
        pltpu.VMEM((CH_PER_TILE, CHUNK), jnp.int32),
        pltpu.VMEM((CH_PER_TILE, CHUNK), jnp.int32),
        pltpu.VMEM((CHUNK, 16), jnp.float32),
        pltpu.VMEM_SHARED((N_PAD, 16), jnp.float32),
        pltpu.VMEM_SHARED((N_PAD, 16), jnp.float32),
    ],
)
def _sc_degrees(src_hbm, dst_hbm, out_hbm,
                src_v, dst_v, fill_v, ds_sh, dd_sh):
    c = lax.axis_index("c")
    s = lax.axis_index("s")
    wid = c * 16 + s

    pltpu.sync_copy(src_hbm.at[pl.ds(wid * CH_PER_TILE, CH_PER_TILE)], src_v)
    pltpu.sync_copy(dst_hbm.at[pl.ds(wid * CH_PER_TILE, CH_PER_TILE)], dst_v)

    def _fill(val):
        vec = jnp.full((16,), val, jnp.float32)

        def _frow(r, _):
            fill_v[r, pl.ds(0, 16)] = vec
            return 0

        lax.fori_loop(0, CHUNK, _frow, 0)

    _fill(0.0)
    for z in range(ROWS_PER_TILE // CHUNK):
        r0 = s * ROWS_PER_TILE + z * CHUNK
        pltpu.sync_copy(fill_v, ds_sh.at[pl.ds(r0, CHUNK)])
        pltpu.sync_copy(fill_v, dd_sh.at[pl.ds(r0, CHUNK)])
    _fill(1.0)
    plsc.subcore_barrier()

    def _edge_chunk(j, _):
        pltpu.sync_copy(fill_v, ds_sh.at[src_v.at[j]], add=True)
        pltpu.sync_copy(fill_v, dd_sh.at[dst_v.at[j]], add=True)
        return 0

    lax.fori_loop(0, CH_PER_TILE, _edge_chunk, 0)
    plsc.subcore_barrier()

    rows = pl.ds(s * ROWS_PER_TILE, ROWS_PER_TILE)
    pltpu.sync_copy(ds_sh.at[rows], out_hbm.at[c, 0, rows])
    pltpu.sync_copy(dd_sh.at[rows], out_hbm.at[c, 1, rows])


def _make_sc_agg(n_tab):
    w = 128

    @functools.partial(
        pl.kernel,
        out_type=jax.ShapeDtypeStruct((n_tab, 2, N_PAD, w), jnp.float32),
        mesh=_MESH,
        scratch_types=[
            pltpu.VMEM((CH_PER_TILE, CHUNK), jnp.int32),
            pltpu.VMEM((CH_PER_TILE, CHUNK), jnp.int32),
            pltpu.VMEM((NBUF, CHUNK, w), jnp.float32),
            pltpu.VMEM_SHARED((N_PAD, w), jnp.float32),
            pltpu.SemaphoreType.DMA((NBUF,)),
            pltpu.SemaphoreType.DMA((NBUF,)),
        ],
    )
    def _sc_agg(h_hbm, src_hbm, dst_hbm, out_hbm,
                src_v, dst_v, rows_v, acc_sh, gsems, ssems):
        c = lax.axis_index("c")
        s = lax.axis_index("s")
        wid = c * 16 + s

        pltpu.sync_copy(src_hbm.at[pl.ds(wid * CH_PER_TILE, CH_PER_TILE)],
                        src_v)
        pltpu.sync_copy(dst_hbm.at[pl.ds(wid * CH_PER_TILE, CH_PER_TILE)],
                        dst_v)

        zero16 = jnp.zeros((16,), jnp.float32)

        def _zrow(r, _):
            def _zcol(q, _):
                rows_v[0, r, pl.ds(q * 16, 16)] = zero16
                return 0
            lax.fori_loop(0, w // 16, _zcol, 0)
            return 0

        lax.fori_loop(0, CHUNK, _zrow, 0)

        my_rows = pl.ds(s * ROWS_PER_TILE, ROWS_PER_TILE)

        for t in range(n_tab):
            for z in range(ROWS_PER_TILE // CHUNK):
                r0 = s * ROWS_PER_TILE + z * CHUNK
                pltpu.sync_copy(rows_v.at[0], acc_sh.at[pl.ds(r0, CHUNK)])
            plsc.subcore_barrier()

            def _burst(base, nb):
                gd = [pltpu.async_copy(h_hbm.at[t].at[src_v.at[base + b]],
                                       rows_v.at[b], gsems.at[b])
                      for b in range(nb)]
                sd = []
                for b in range(nb):
                    gd[b].wait()
                    sd.append(pltpu.async_copy(
                        rows_v.at[b], acc_sh.at[dst_v.at[base + b]],
                        ssems.at[b], add=True))
                for b in range(nb):
                    sd[b].wait()

            def _step(st, _):
                _burst(st * NBUF, NBUF)
                return 0

            n_steps = CH_PER_TILE // NBUF
            lax.fori_loop(0, n_steps, _step, 0)
            if CH_PER_TILE % NBUF:
                _burst(n_steps * NBUF, CH_PER_TILE % NBUF)
            plsc.subcore_barrier()

            pltpu.sync_copy(acc_sh.at[my_rows], out_hbm.at[t, c, my_rows])
            if t + 1 < n_tab:
                plsc.subcore_barrier()

            if t + 1 < n_tab:
                lax.fori_loop(0, CHUNK, _zrow, 0)

    return _sc_agg


_sc_agg1 = _make_sc_agg(1)
_sc_agg3 = _make_sc_agg(3)


BN = 1000
GRID = N // BN

_row_spec = lambda wdt: pl.BlockSpec((BN, wdt), lambda i: (i, 0))
_vec_spec = pl.BlockSpec((BN, 1), lambda i: (i, 0))
_h1_spec = pl.BlockSpec((1, BN, D_IN), lambda i: (0, i, 0))
_y2_spec = pl.BlockSpec((3, BN, 128), lambda i: (0, i, 0))
_p2_spec = pl.BlockSpec((2, BN, 128), lambda i: (0, i, 0))
_p32_spec = pl.BlockSpec((3, 2, BN, 128), lambda i: (0, 0, i, 0))
_deg_spec = pl.BlockSpec((2, BN, 1), lambda i: (0, i, 0))


def _full(shape):
    nd = len(shape)
    return pl.BlockSpec(shape, lambda i: (0,) * nd)


def _tc0_body(f_ref, od_ref, id_ref, wres_ref, bres_ref,
              ns_ref, nd_ref, h1_ref, res_ref):
    od = od_ref[0] + od_ref[1]
    ig = id_ref[0] + id_ref[1]
    ns = lax.rsqrt(jnp.where(od > 0, od, 1.0))
    nd = lax.rsqrt(jnp.where(ig > 0, ig, 1.0))
    ns_ref[...] = ns
    nd_ref[...] = nd
    f = f_ref[...]
    h1_ref[0] = f * ns
    r = jnp.dot(f, wres_ref[...], preferred_element_type=jnp.float32)
    res_ref[...] = _elu(r + bres_ref[...][None, :])


def _tc0(f, od2, id2, Wres, bres):
    return pl.pallas_call(
        _tc0_body,
        grid=(GRID,),
        in_specs=[_row_spec(D_IN), _deg_spec, _deg_spec,
                  _full((D_IN, D_OUT)), _full((D_OUT,))],
        out_specs=[_vec_spec, _vec_spec, _h1_spec, _row_spec(D_OUT)],
        out_shape=[
            jax.ShapeDtypeStruct((N, 1), jnp.float32),
            jax.ShapeDtypeStruct((N, 1), jnp.float32),
            jax.ShapeDtypeStruct((1, N, D_IN), jnp.float32),
            jax.ShapeDtypeStruct((N, D_OUT), jnp.float32),
        ],
    )(f, od2, id2, Wres, bres)


def _tc1_body(p_ref, nd_ref, ns_ref, w1_ref, b1_ref, w2s_ref, y2_ref):
    a1 = (p_ref[0] + p_ref[1]) * nd_ref[...]
    x1 = _elu(jnp.dot(a1, w1_ref[...], preferred_element_type=jnp.float32)
              + b1_ref[...][None, :])
    x1n = x1 * ns_ref[...]
    for t in range(3):
        y2_ref[t] = jnp.dot(x1n, w2s_ref[t],
                            preferred_element_type=jnp.float32)


def _tc1(p1, nd, ns, W1, b1, W2s):
    return pl.pallas_call(
        _tc1_body,
        grid=(GRID,),
        in_specs=[_p2_spec, _vec_spec, _vec_spec,
                  _full((D_IN, H1)), _full((H1,)), _full((3, H1, 128))],
        out_specs=[_y2_spec],
        out_shape=[jax.ShapeDtypeStruct((3, N, 128), jnp.float32)],
    )(p1, nd, ns, W1, b1, W2s)[0]


def _tc2_body(p_ref, nd_ref, ns_ref, b2s_ref, w3s_ref, y3_ref):
    nd = nd_ref[...]
    ns = ns_ref[...]
    acc = None
    for t in range(3):
        x2 = _elu((p_ref[t, 0] + p_ref[t, 1]) * nd + b2s_ref[t][None, :])
        d = jnp.dot(x2 * ns, w3s_ref[t], preferred_element_type=jnp.float32)
        acc = d if acc is None else acc + d
    y3_ref[0] = acc


def _tc2(p2, nd, ns, b2s, W3s):
    return pl.pallas_call(
        _tc2_body,
        grid=(GRID,),
        in_specs=[_p32_spec, _vec_spec, _vec_spec,
                  _full((3, 128)), _full((3, 128, D_OUT))],
        out_specs=[_h1_spec],
        out_shape=[jax.ShapeDtypeStruct((1, N, D_OUT), jnp.float32)],
    )(p2, nd, ns, b2s, W3s)[0]


def _tc3_body(p_ref, nd_ref, b3_ref, out_ref):
    out_ref[...] = ((p_ref[0] + p_ref[1]) * nd_ref[...]
                    + b3_ref[...][None, :])


def _tc3(p3, nd, b3):
    return pl.pallas_call(
        _tc3_body,
        grid=(GRID,),
        in_specs=[_p2_spec, _vec_spec, _full((D_OUT,))],
        out_specs=[_row_spec(D_OUT)],
        out_shape=[jax.ShapeDtypeStruct((N, D_OUT), jnp.float32)],
    )(p3, nd, b3)[0]


def kernel(features, edge_index, W1, b1, W2, b2, W3, b3, Wres, bres):
    pad_e = E_PAD - E
    dummy = N + (jnp.arange(pad_e, dtype=jnp.int32) % (N_PAD - N))
    src_r = edge_index[0].astype(jnp.int32)
    dst_r = edge_index[1].astype(jnp.int32)
    shape2 = (E_PAD // CHUNK, CHUNK)
    src_deg = jnp.concatenate([src_r, dummy]).reshape(shape2)
    src_agg = jnp.concatenate(
        [src_r, jnp.arange(pad_e, dtype=jnp.int32) % N]).reshape(shape2)
    dst = jnp.concatenate([dst_r, dummy]).reshape(shape2)

    deg = _sc_degrees(src_deg, dst)
    od2 = deg[:, 0, :, :1]
    id2 = deg[:, 1, :, :1]

    ns, nd, h1, res = _tc0(features, od2, id2, Wres, bres)

    p1 = _sc_agg1(h1, src_agg, dst)
    W2s = jnp.stack([W2[:, :128], W2[:, 128:256],
                     jnp.pad(W2[:, 256:], ((0, 0), (0, 64)))])
    y2 = _tc1(p1[0], nd, ns, W1, b1, W2s)

    p2 = _sc_agg3(y2, src_agg, dst)
    b2s = jnp.stack([b2[:128], b2[128:256], jnp.pad(b2[256:], (0, 64))])
    W3s = jnp.stack([W3[:128], W3[128:256],
                     jnp.pad(W3[256:], ((0, 64), (0, 0)))])
    y3 = _tc2(p2, nd, ns, b2s, W3s)

    p3 = _sc_agg1(y3, src_agg, dst)
    x = _tc3(p3[0], nd, b3)
    return (x, res)


# speedup vs baseline: 1.0810x; 1.0410x over previous
import functools

import jax
import jax.numpy as jnp
from jax import lax
from jax.experimental import pallas as pl
from jax.experimental.pallas import tpu as pltpu
from jax.experimental.pallas import tpu_sc as plsc

N = 10000
E = 160000
D_IN = 128
H1 = 640
H2 = 320
D_OUT = 128

N_PAD = 10240
E_PAD = 163840
CHUNK = 128
CH_PER_TILE = 40
EPT = CHUNK * CH_PER_TILE
ROWS_PER_TILE = N_PAD // 16
NBUF = 2

_MESH = plsc.VectorSubcoreMesh(core_axis_name="c", subcore_axis_name="s")


def _elu(v):
    return jnp.where(v > 0, v, jnp.exp(v) - 1.0)


@functools.partial(
    pl.kernel,
    out_type=jax.ShapeDtypeStruct((2, 2, N_PAD, 16), jnp.float32),
    mesh=_MESH,
    compiler_params=pltpu.CompilerParams(use_tc_tiling_on_sc=False),
    scratch_types=[
        pltpu.VMEM((CH_PER_TILE, CHUNK), jnp.int32),
        pltpu.VMEM((CH_PER_TILE, CHUNK), jnp.int32),
        pltpu.VMEM((CHUNK, 16), jnp.float32),
        pltpu.VMEM_SHARED((N_PAD, 16), jnp.float32),
        pltpu.VMEM_SHARED((N_PAD, 16), jnp.float32),
    ],
)
def _sc_degrees(src_hbm, dst_hbm, out_hbm,
                src_v, dst_v, fill_v, ds_sh, dd_sh):
    c = lax.axis_index("c")
    s = lax.axis_index("s")
    wid = c * 16 + s

    pltpu.sync_copy(src_hbm.at[pl.ds(wid * CH_PER_TILE, CH_PER_TILE)], src_v)
    pltpu.sync_copy(dst_hbm.at[pl.ds(wid * CH_PER_TILE, CH_PER_TILE)], dst_v)

    def _fill(val):
        vec = jnp.full((16,), val, jnp.float32)

        def _frow(r, _):
            fill_v[r, pl.ds(0, 16)] = vec
            return 0

        lax.fori_loop(0, CHUNK, _frow, 0)

    _fill(0.0)
    for z in range(ROWS_PER_TILE // CHUNK):
        r0 = s * ROWS_PER_TILE + z * CHUNK
        pltpu.sync_copy(fill_v, ds_sh.at[pl.ds(r0, CHUNK)])
        pltpu.sync_copy(fill_v, dd_sh.at[pl.ds(r0, CHUNK)])
    _fill(1.0)
    plsc.subcore_barrier()

    def _edge_chunk(j, _):
        pltpu.sync_copy(fill_v, ds_sh.at[src_v.at[j]], add=True)
        pltpu.sync_copy(fill_v, dd_sh.at[dst_v.at[j]], add=True)
        return 0

    lax.fori_loop(0, CH_PER_TILE, _edge_chunk, 0)
    plsc.subcore_barrier()

    rows = pl.ds(s * ROWS_PER_TILE, ROWS_PER_TILE)
    pltpu.sync_copy(ds_sh.at[rows], out_hbm.at[c, 0, rows])
    pltpu.sync_copy(dd_sh.at[rows], out_hbm.at[c, 1, rows])


def _make_sc_agg(n_tab, w=128, tiled=True):
    params = (pltpu.CompilerParams() if tiled
              else pltpu.CompilerParams(use_tc_tiling_on_sc=False))

    @functools.partial(
        pl.kernel,
        out_type=jax.ShapeDtypeStruct((n_tab, 2, N_PAD, w), jnp.float32),
        mesh=_MESH,
        compiler_params=params,
        scratch_types=[
            pltpu.VMEM((CH_PER_TILE, CHUNK), jnp.int32),
            pltpu.VMEM((CH_PER_TILE, CHUNK), jnp.int32),
            pltpu.VMEM((NBUF, CHUNK, w), jnp.float32),
            pltpu.VMEM_SHARED((N_PAD, w), jnp.float32),
            pltpu.SemaphoreType.DMA((NBUF,)),
            pltpu.SemaphoreType.DMA((NBUF,)),
        ],
    )
    def _sc_agg(h_hbm, src_hbm, dst_hbm, out_hbm,
                src_v, dst_v, rows_v, acc_sh, gsems, ssems):
        c = lax.axis_index("c")
        s = lax.axis_index("s")
        wid = c * 16 + s

        pltpu.sync_copy(src_hbm.at[pl.ds(wid * CH_PER_TILE, CH_PER_TILE)],
                        src_v)
        pltpu.sync_copy(dst_hbm.at[pl.ds(wid * CH_PER_TILE, CH_PER_TILE)],
                        dst_v)

        zero16 = jnp.zeros((16,), jnp.float32)

        def _zrow(r, _):
            def _zcol(q, _):
                rows_v[0, r, pl.ds(q * 16, 16)] = zero16
                return 0
            lax.fori_loop(0, w // 16, _zcol, 0)
            return 0

        lax.fori_loop(0, CHUNK, _zrow, 0)

        my_rows = pl.ds(s * ROWS_PER_TILE, ROWS_PER_TILE)

        for t in range(n_tab):
            for z in range(ROWS_PER_TILE // CHUNK):
                r0 = s * ROWS_PER_TILE + z * CHUNK
                pltpu.sync_copy(rows_v.at[0], acc_sh.at[pl.ds(r0, CHUNK)])
            plsc.subcore_barrier()

            def _burst(base, nb):
                gd = [pltpu.async_copy(h_hbm.at[t].at[src_v.at[base + b]],
                                       rows_v.at[b], gsems.at[b])
                      for b in range(nb)]
                sd = []
                for b in range(nb):
                    gd[b].wait()
                    sd.append(pltpu.async_copy(
                        rows_v.at[b], acc_sh.at[dst_v.at[base + b]],
                        ssems.at[b], add=True))
                for b in range(nb):
                    sd[b].wait()

            def _step(st, _):
                _burst(st * NBUF, NBUF)
                return 0

            n_steps = CH_PER_TILE // NBUF
            lax.fori_loop(0, n_steps, _step, 0)
            if CH_PER_TILE % NBUF:
                _burst(n_steps * NBUF, CH_PER_TILE % NBUF)
            plsc.subcore_barrier()

            pltpu.sync_copy(acc_sh.at[my_rows], out_hbm.at[t, c, my_rows])
            if t + 1 < n_tab:
                plsc.subcore_barrier()

            if t + 1 < n_tab:
                lax.fori_loop(0, CHUNK, _zrow, 0)

    return _sc_agg


_sc_agg1 = _make_sc_agg(1)
_sc_agg2 = _make_sc_agg(2)
_sc_agg64 = _make_sc_agg(1, w=64, tiled=False)


BN = 1000
GRID = N // BN

_row_spec = lambda wdt: pl.BlockSpec((BN, wdt), lambda i: (i, 0))
_vec_spec = pl.BlockSpec((BN, 1), lambda i: (i, 0))
_h1_spec = pl.BlockSpec((1, BN, D_IN), lambda i: (0, i, 0))
_y2_spec = pl.BlockSpec((2, BN, 128), lambda i: (0, i, 0))
_p2_spec = pl.BlockSpec((2, BN, 128), lambda i: (0, i, 0))
_p22_spec = pl.BlockSpec((2, 2, BN, 128), lambda i: (0, 0, i, 0))
_pc_spec = pl.BlockSpec((2, BN, 64), lambda i: (0, i, 0))
_deg_spec = pl.BlockSpec((2, BN, 1), lambda i: (0, i, 0))


def _full(shape):
    nd = len(shape)
    return pl.BlockSpec(shape, lambda i: (0,) * nd)


def _tc0_body(f_ref, od_ref, id_ref, wres_ref, bres_ref,
              ns_ref, nd_ref, h1_ref, res_ref):
    od = od_ref[0] + od_ref[1]
    ig = id_ref[0] + id_ref[1]
    ns = lax.rsqrt(jnp.where(od > 0, od, 1.0))
    nd = lax.rsqrt(jnp.where(ig > 0, ig, 1.0))
    ns_ref[...] = ns
    nd_ref[...] = nd
    f = f_ref[...]
    h1_ref[0] = f * ns
    r = jnp.dot(f, wres_ref[...], preferred_element_type=jnp.float32)
    res_ref[...] = _elu(r + bres_ref[...][None, :])


def _tc0(f, od2, id2, Wres, bres):
    return pl.pallas_call(
        _tc0_body,
        grid=(GRID,),
        in_specs=[_row_spec(D_IN), _deg_spec, _deg_spec,
                  _full((D_IN, D_OUT)), _full((D_OUT,))],
        out_specs=[_vec_spec, _vec_spec, _h1_spec, _row_spec(D_OUT)],
        out_shape=[
            jax.ShapeDtypeStruct((N, 1), jnp.float32),
            jax.ShapeDtypeStruct((N, 1), jnp.float32),
            jax.ShapeDtypeStruct((1, N, D_IN), jnp.float32),
            jax.ShapeDtypeStruct((N, D_OUT), jnp.float32),
        ],
    )(f, od2, id2, Wres, bres)


def _tc1_body(p_ref, nd_ref, ns_ref, w1_ref, b1_ref, w2s_ref, w2c_ref,
              y2_ref, y2c_ref):
    a1 = (p_ref[0] + p_ref[1]) * nd_ref[...]
    x1 = _elu(jnp.dot(a1, w1_ref[...], preferred_element_type=jnp.float32)
              + b1_ref[...][None, :])
    x1n = x1 * ns_ref[...]
    for t in range(2):
        y2_ref[t] = jnp.dot(x1n, w2s_ref[t],
                            preferred_element_type=jnp.float32)
    y2c_ref[0] = jnp.dot(x1n, w2c_ref[...],
                         preferred_element_type=jnp.float32)


def _tc1(p1, nd, ns, W1, b1, W2s, W2c):
    return pl.pallas_call(
        _tc1_body,
        grid=(GRID,),
        in_specs=[_p2_spec, _vec_spec, _vec_spec,
                  _full((D_IN, H1)), _full((H1,)), _full((2, H1, 128)),
                  _full((H1, 64))],
        out_specs=[_y2_spec, pl.BlockSpec((1, BN, 64), lambda i: (0, i, 0))],
        out_shape=[jax.ShapeDtypeStruct((2, N, 128), jnp.float32),
                   jax.ShapeDtypeStruct((1, N, 64), jnp.float32)],
    )(p1, nd, ns, W1, b1, W2s, W2c)


def _tc2_body(p_ref, pc_ref, nd_ref, ns_ref, b2s_ref, b2c_ref,
              w3s_ref, w3c_ref, y3_ref):
    nd = nd_ref[...]
    ns = ns_ref[...]
    acc = None
    for t in range(2):
        x2 = _elu((p_ref[t, 0] + p_ref[t, 1]) * nd + b2s_ref[t][None, :])
        d = jnp.dot(x2 * ns, w3s_ref[t], preferred_element_type=jnp.float32)
        acc = d if acc is None else acc + d
    x2c = _elu((pc_ref[0] + pc_ref[1]) * nd + b2c_ref[...][None, :])
    y3_ref[0] = acc + jnp.dot(x2c * ns, w3c_ref[...],
                              preferred_element_type=jnp.float32)


def _tc2(p2, p2c, nd, ns, b2s, b2c, W3s, W3c):
    return pl.pallas_call(
        _tc2_body,
        grid=(GRID,),
        in_specs=[_p22_spec, _pc_spec, _vec_spec, _vec_spec,
                  _full((2, 128)), _full((64,)),
                  _full((2, 128, D_OUT)), _full((64, D_OUT))],
        out_specs=[_h1_spec],
        out_shape=[jax.ShapeDtypeStruct((1, N, D_OUT), jnp.float32)],
    )(p2, p2c, nd, ns, b2s, b2c, W3s, W3c)[0]


def _tc3_body(p_ref, nd_ref, b3_ref, out_ref):
    out_ref[...] = ((p_ref[0] + p_ref[1]) * nd_ref[...]
                    + b3_ref[...][None, :])


def _tc3(p3, nd, b3):
    return pl.pallas_call(
        _tc3_body,
        grid=(GRID,),
        in_specs=[_p2_spec, _vec_spec, _full((D_OUT,))],
        out_specs=[_row_spec(D_OUT)],
        out_shape=[jax.ShapeDtypeStruct((N, D_OUT), jnp.float32)],
    )(p3, nd, b3)[0]


def kernel(features, edge_index, W1, b1, W2, b2, W3, b3, Wres, bres):
    pad_e = E_PAD - E
    dummy = N + (jnp.arange(pad_e, dtype=jnp.int32) % (N_PAD - N))
    src_r = edge_index[0].astype(jnp.int32)
    dst_r = edge_index[1].astype(jnp.int32)
    shape2 = (E_PAD // CHUNK, CHUNK)
    src_deg = jnp.concatenate([src_r, dummy]).reshape(shape2)
    src_agg = jnp.concatenate(
        [src_r, jnp.arange(pad_e, dtype=jnp.int32) % N]).reshape(shape2)
    dst = jnp.concatenate([dst_r, dummy]).reshape(shape2)

    deg = _sc_degrees(src_deg, dst)
    od2 = deg[:, 0, :, :1]
    id2 = deg[:, 1, :, :1]

    ns, nd, h1, res = _tc0(features, od2, id2, Wres, bres)

    p1 = _sc_agg1(h1, src_agg, dst)
    W2s = jnp.stack([W2[:, :128], W2[:, 128:256]])
    y2, y2c = _tc1(p1[0], nd, ns, W1, b1, W2s, W2[:, 256:])

    p2 = _sc_agg2(y2, src_agg, dst)
    p2c = _sc_agg64(y2c, src_agg, dst)
    b2s = jnp.stack([b2[:128], b2[128:256]])
    W3s = jnp.stack([W3[:128], W3[128:256]])
    y3 = _tc2(p2, p2c[0], nd, ns, b2s, b2[256:], W3s, W3[256:])

    p3 = _sc_agg1(y3, src_agg, dst)
    x = _tc3(p3[0], nd, b3)
    return (x, res)
